# Initial kernel scaffold; baseline (speedup 1.0000x reference)
import jax
import jax.numpy as jnp
from jax.experimental import pallas as pl


def kernel(x, edge_index, W_ih, W_hh, b_ih, b_hh, W_gcn, b_gcn, W_fc, b_fc):
    raise NotImplementedError("write your pallas kernel here")


# speedup vs baseline: 18.0381x; 18.0381x over previous
import functools

import jax
import jax.numpy as jnp
from jax import lax
from jax.experimental import pallas as pl
from jax.experimental.pallas import tpu as pltpu
from jax.experimental.pallas import tpu_sc as plsc

N = 10000
E = 320000
T = 5
D = 128
H = 128
O = 128

NC = 2
NS = 16
NW = NC * NS
NP = 10240
ROWS_PER_TILE = NP // NS
CHUNK = 128
NCHUNKS = E // CHUNK
CH_FULL = NCHUNKS // NW
CH_REM = NCHUNKS % NW

@functools.cache
def _mesh():
    return plsc.VectorSubcoreMesh(core_axis_name="c", subcore_axis_name="s",
                                  num_cores=NC, num_subcores=NS)


def _deg_body(dst_hbm, zeros_hbm, deg_out, idx_v, ones_v, deg_sh):
    c = lax.axis_index("c")
    s = lax.axis_index("s")
    wid = s * NC + c
    pltpu.sync_copy(zeros_hbm, deg_sh.at[pl.ds(s * ROWS_PER_TILE, ROWS_PER_TILE)])
    for i in range(CHUNK // 16):
        ones_v[pl.ds(i * 16, 16)] = jnp.full((16,), 1.0, jnp.float32)
    plsc.subcore_barrier()

    def body(j, carry):
        base = (j * NW + wid) * CHUNK
        pltpu.sync_copy(dst_hbm.at[pl.ds(base, CHUNK)], idx_v)
        pltpu.sync_copy(ones_v, deg_sh.at[idx_v], add=True)
        return carry

    lax.fori_loop(0, CH_FULL, body, 0)

    @pl.when(wid < CH_REM)
    def _():
        body(CH_FULL, 0)

    plsc.subcore_barrier()
    pltpu.sync_copy(deg_sh.at[pl.ds(s * ROWS_PER_TILE, ROWS_PER_TILE)],
                    deg_out.at[c, pl.ds(s * ROWS_PER_TILE, ROWS_PER_TILE)])


@functools.cache
def _deg_call():
    return pl.kernel(
        _deg_body,
        out_type=jax.ShapeDtypeStruct((NC, NP), jnp.float32),
        mesh=_mesh(),
        scratch_types=[
            pltpu.VMEM((CHUNK,), jnp.int32),
            pltpu.VMEM((CHUNK,), jnp.float32),
            pltpu.VMEM_SHARED((NP,), jnp.float32),
        ],
    )


def _agg_body(src_hbm, dst_hbm, y_hbm, zeros_hbm, agg_out,
              sidx_v, didx_v, rows_v, agg_sh, sem):
    c = lax.axis_index("c")
    s = lax.axis_index("s")
    wid = s * NC + c
    pltpu.sync_copy(zeros_hbm, agg_sh.at[pl.ds(s * ROWS_PER_TILE, ROWS_PER_TILE)])
    plsc.subcore_barrier()

    def body(j, carry):
        base = (j * NW + wid) * CHUNK
        pltpu.sync_copy(src_hbm.at[pl.ds(base, CHUNK)], sidx_v)
        pltpu.sync_copy(dst_hbm.at[pl.ds(base, CHUNK)], didx_v)
        pltpu.async_copy(y_hbm.at[sidx_v], rows_v, sem).wait()
        pltpu.sync_copy(rows_v, agg_sh.at[didx_v], add=True)
        return carry

    lax.fori_loop(0, CH_FULL, body, 0)

    @pl.when(wid < CH_REM)
    def _():
        body(CH_FULL, 0)

    plsc.subcore_barrier()
    pltpu.sync_copy(agg_sh.at[pl.ds(s * ROWS_PER_TILE, ROWS_PER_TILE)],
                    agg_out.at[c, pl.ds(s * ROWS_PER_TILE, ROWS_PER_TILE)])


@functools.cache
def _agg_call():
    return pl.kernel(
        _agg_body,
        out_type=jax.ShapeDtypeStruct((NC, NP, H), jnp.float32),
        mesh=_mesh(),
        scratch_types=[
            pltpu.VMEM((CHUNK,), jnp.int32),
            pltpu.VMEM((CHUNK,), jnp.int32),
            pltpu.VMEM((CHUNK, H), jnp.float32),
            pltpu.VMEM_SHARED((NP, H), jnp.float32),
            pltpu.SemaphoreType.DMA,
        ],
    )


BN = 2000


def _lstm_body(x_ref, wih_ref, whh_ref, bih_ref, bhh_ref, h_out):
    b = bih_ref[...] + bhh_ref[...]
    h = jnp.zeros((BN, H), jnp.float32)
    c = jnp.zeros((BN, H), jnp.float32)
    for t in range(T):
        xt = x_ref[:, t, :]
        gates = (jnp.dot(xt, wih_ref[...], preferred_element_type=jnp.float32)
                 + jnp.dot(h, whh_ref[...], preferred_element_type=jnp.float32)
                 + b)
        i = jax.nn.sigmoid(gates[:, 0:H])
        f = jax.nn.sigmoid(gates[:, H:2 * H])
        g = jnp.tanh(gates[:, 2 * H:3 * H])
        o = jax.nn.sigmoid(gates[:, 3 * H:4 * H])
        c = f * c + i * g
        h = o * jnp.tanh(c)
    h_out[...] = h


def _lstm_call(x, wihT, whhT, bih, bhh):
    return pl.pallas_call(
        _lstm_body,
        grid=(N // BN,),
        in_specs=[
            pl.BlockSpec((BN, T, D), lambda i: (i, 0, 0)),
            pl.BlockSpec((D, 4 * H), lambda i: (0, 0)),
            pl.BlockSpec((H, 4 * H), lambda i: (0, 0)),
            pl.BlockSpec((1, 4 * H), lambda i: (0, 0)),
            pl.BlockSpec((1, 4 * H), lambda i: (0, 0)),
        ],
        out_specs=pl.BlockSpec((BN, H), lambda i: (i, 0)),
        out_shape=jax.ShapeDtypeStruct((N, H), jnp.float32),
    )(x, wihT, whhT, bih, bhh)


def _y_body(h_ref, wg_ref, degp_ref, y_out):
    deg = degp_ref[:, 0] + degp_ref[:, 1] + 1.0
    dinv = lax.rsqrt(deg)
    xw = jnp.dot(h_ref[...], wg_ref[...], preferred_element_type=jnp.float32)
    y_out[...] = xw * dinv[:, None]


def _y_call(h, wgT, degp):
    return pl.pallas_call(
        _y_body,
        grid=(N // BN,),
        in_specs=[
            pl.BlockSpec((BN, H), lambda i: (i, 0)),
            pl.BlockSpec((H, H), lambda i: (0, 0)),
            pl.BlockSpec((BN, NC), lambda i: (i, 0)),
        ],
        out_specs=pl.BlockSpec((BN, H), lambda i: (i, 0)),
        out_shape=jax.ShapeDtypeStruct((N, H), jnp.float32),
    )(h, wgT, degp)


def _fin_body(p_ref, y_ref, degp_ref, bg_ref, wf_ref, bf_ref, out_ref):
    deg = degp_ref[:, 0] + degp_ref[:, 1] + 1.0
    dinv = lax.rsqrt(deg)
    aggr = p_ref[0] + p_ref[1] + y_ref[...]
    agg = aggr * dinv[:, None] + bg_ref[...]
    r = jnp.maximum(agg, 0.0)
    out_ref[...] = jnp.dot(r, wf_ref[...], preferred_element_type=jnp.float32) + bf_ref[...]


def _fin_call(p, y, degp, bg, wfT, bf):
    return pl.pallas_call(
        _fin_body,
        grid=(N // BN,),
        in_specs=[
            pl.BlockSpec((NC, BN, H), lambda i: (0, i, 0)),
            pl.BlockSpec((BN, H), lambda i: (i, 0)),
            pl.BlockSpec((BN, NC), lambda i: (i, 0)),
            pl.BlockSpec((1, H), lambda i: (0, 0)),
            pl.BlockSpec((H, O), lambda i: (0, 0)),
            pl.BlockSpec((1, O), lambda i: (0, 0)),
        ],
        out_specs=pl.BlockSpec((BN, O), lambda i: (i, 0)),
        out_shape=jax.ShapeDtypeStruct((N, O), jnp.float32),
    )(p, y, degp, bg, wfT, bf)


def kernel(x, edge_index, W_ih, W_hh, b_ih, b_hh, W_gcn, b_gcn, W_fc, b_fc):
    src = edge_index[0]
    dst = edge_index[1]
    zrow = jnp.zeros((ROWS_PER_TILE,), jnp.float32)
    zblk = jnp.zeros((ROWS_PER_TILE, H), jnp.float32)

    degp = _deg_call()(dst, zrow)
    degpT = degp[:, :N].T
    h = _lstm_call(x, W_ih.T, W_hh.T, b_ih[None, :], b_hh[None, :])
    y = _y_call(h, W_gcn.T, degpT)
    aggp = _agg_call()(src, dst, y, zblk)
    out = _fin_call(aggp[:, :N, :], y, degpT, b_gcn[None, :], W_fc.T, b_fc[None, :])
    return out
